# baseline (speedup 1.0000x reference)
import jax
import jax.numpy as jnp
from jax.experimental import pallas as pl
from jax.experimental.pallas import tpu as pltpu

N = 16384
K = 16
KA = 24
TI = 2048
TJC = 512
NI = N // TI
NJC = N // TJC
SLAB = 64


def _body(a_ref, t_ref, out_ref, at_ref, b_ref):
    i = pl.program_id(0)

    @pl.when(i == 0)
    def _():
        t = t_ref[...]
        bn = jnp.sum(t * t, axis=1, keepdims=True)
        b_ref[...] = jnp.concatenate(
            [-2.0 * t, bn, jnp.zeros((N, KA - K - 1), jnp.float32)], axis=1)
        a = a_ref[...]
        at_ref[...] = jnp.concatenate(
            [a.T, jnp.ones((1, N), jnp.float32),
             jnp.zeros((KA - K - 1, N), jnp.float32)], axis=0)

    at = at_ref[:, pl.ds(i * TI, TI)]
    acc = jnp.full((SLAB, TI), jnp.inf, jnp.float32)
    for c in range(NJC):
        b = b_ref[c * TJC:(c + 1) * TJC, :]
        val = jax.lax.dot_general(b, at, (((1,), (0,)), ((), ())),
                                  preferred_element_type=jnp.float32)
        parts = [val[r * SLAB:(r + 1) * SLAB, :] for r in range(TJC // SLAB)]
        while len(parts) > 1:
            nxt = [jnp.minimum(parts[t], parts[t + 1])
                   for t in range(0, len(parts) - 1, 2)]
            if len(parts) % 2:
                nxt.append(parts[-1])
            parts = nxt
        acc = jnp.minimum(acc, parts[0])

    mrow = jnp.min(acc, axis=0, keepdims=True)
    a_sq = at[:K, :]
    s = (jnp.sum(mrow) + jnp.sum(a_sq * a_sq)) * (1.0 / N)

    @pl.when(i == 0)
    def _():
        out_ref[0, 0] = s

    @pl.when(i > 0)
    def _():
        out_ref[0, 0] += s


@jax.jit
def kernel(outputs, targets):
    res = pl.pallas_call(
        _body,
        grid=(NI,),
        in_specs=[
            pl.BlockSpec((N, K), lambda i: (0, 0)),
            pl.BlockSpec((N, K), lambda i: (0, 0)),
        ],
        out_specs=pl.BlockSpec(
            (1, 1), lambda i: (0, 0), memory_space=pltpu.SMEM),
        out_shape=jax.ShapeDtypeStruct((1, 1), jnp.float32),
        scratch_shapes=[
            pltpu.VMEM((KA, N), jnp.float32),
            pltpu.VMEM((N, KA), jnp.float32),
        ],
        compiler_params=pltpu.CompilerParams(
            dimension_semantics=("arbitrary",)),
    )(outputs, targets)
    return res[0, 0]


# speedup vs baseline: 1.0012x; 1.0012x over previous
import jax
import jax.numpy as jnp
from jax.experimental import pallas as pl
from jax.experimental.pallas import tpu as pltpu

N = 16384
K = 16
KA = 24
TI = 2048
TJC = 2048
NI = N // TI
NJC = N // TJC


def _fold_min(val):
    parts = [val[:, k * 128:(k + 1) * 128] for k in range(TJC // 128)]
    while len(parts) > 1:
        nxt = [jnp.minimum(parts[t], parts[t + 1])
               for t in range(0, len(parts) - 1, 2)]
        if len(parts) % 2:
            nxt.append(parts[-1])
        parts = nxt
    return parts[0]


def _body(a_ref, t_ref, out_ref, bt_ref, aa_ref):
    i = pl.program_id(0)

    @pl.when(i == 0)
    def _():
        t = t_ref[...]
        bt = t.T
        bn = jnp.sum(bt * bt, axis=0, keepdims=True)
        bt_ref[...] = jnp.concatenate(
            [-2.0 * bt, bn, jnp.zeros((KA - K - 1, N), jnp.float32)], axis=0)
        a_all = a_ref[...]
        aa_ref[...] = jnp.concatenate(
            [a_all, jnp.ones((N, 1), jnp.float32),
             jnp.zeros((N, KA - K - 1), jnp.float32)], axis=1)

    a = a_ref[pl.ds(i * TI, TI), :]
    a_aug = aa_ref[pl.ds(i * TI, TI), :]

    acc = None
    for c in range(NJC):
        bt = bt_ref[:, c * TJC:(c + 1) * TJC]
        val = jax.lax.dot_general(a_aug, bt, (((1,), (0,)), ((), ())),
                                  preferred_element_type=jnp.float32)
        m = _fold_min(val)
        acc = m if acc is None else jnp.minimum(acc, m)

    an = jnp.sum(a * a, axis=1, keepdims=True)
    row = jnp.min(acc, axis=1, keepdims=True) + an
    s = jnp.sum(row) * (1.0 / N)

    @pl.when(i == 0)
    def _():
        out_ref[0, 0] = s

    @pl.when(i > 0)
    def _():
        out_ref[0, 0] += s


@jax.jit
def kernel(outputs, targets):
    res = pl.pallas_call(
        _body,
        grid=(NI,),
        in_specs=[
            pl.BlockSpec((N, K), lambda i: (0, 0)),
            pl.BlockSpec((N, K), lambda i: (0, 0)),
        ],
        out_specs=pl.BlockSpec(
            (1, 1), lambda i: (0, 0), memory_space=pltpu.SMEM),
        out_shape=jax.ShapeDtypeStruct((1, 1), jnp.float32),
        scratch_shapes=[
            pltpu.VMEM((KA, N), jnp.float32),
            pltpu.VMEM((N, KA), jnp.float32),
        ],
        compiler_params=pltpu.CompilerParams(
            dimension_semantics=("arbitrary",)),
    )(outputs, targets)
    return res[0, 0]


# speedup vs baseline: 1.0274x; 1.0261x over previous
import jax
import jax.numpy as jnp
from jax.experimental import pallas as pl
from jax.experimental.pallas import tpu as pltpu

N = 16384
K = 16
KA = 24
TI = 2048
TJC = 2048
NI = N // TI
NJC = N // TJC


def _fold_min(val):
    parts = [val[:, k * 128:(k + 1) * 128] for k in range(TJC // 128)]
    while len(parts) > 1:
        nxt = [jnp.minimum(parts[t], parts[t + 1])
               for t in range(0, len(parts) - 1, 2)]
        if len(parts) % 2:
            nxt.append(parts[-1])
        parts = nxt
    return parts[0]


def _body(a_ref, t_ref, out_ref, bt_ref):
    i = pl.program_id(0)

    @pl.when(i == 0)
    def _():
        t = t_ref[...]
        bt = t.T
        bn = jnp.sum(bt * bt, axis=0, keepdims=True)
        bt_ref[...] = jnp.concatenate(
            [-2.0 * bt, bn, jnp.zeros((KA - K - 1, N), jnp.float32)], axis=0)

    a = a_ref[...]
    a_aug = jnp.concatenate(
        [a, jnp.ones((TI, 1), jnp.float32),
         jnp.zeros((TI, KA - K - 1), jnp.float32)], axis=1)

    acc = None
    for c in range(NJC):
        bt = bt_ref[:, c * TJC:(c + 1) * TJC]
        val = jax.lax.dot_general(a_aug, bt, (((1,), (0,)), ((), ())),
                                  preferred_element_type=jnp.float32)
        m = _fold_min(val)
        acc = m if acc is None else jnp.minimum(acc, m)

    an = jnp.sum(a * a, axis=1, keepdims=True)
    row = jnp.min(acc, axis=1, keepdims=True) + an
    s = jnp.sum(row) * (1.0 / N)

    @pl.when(i == 0)
    def _():
        out_ref[0, 0] = s

    @pl.when(i > 0)
    def _():
        out_ref[0, 0] += s


@jax.jit
def kernel(outputs, targets):
    res = pl.pallas_call(
        _body,
        grid=(NI,),
        in_specs=[
            pl.BlockSpec((TI, K), lambda i: (i, 0)),
            pl.BlockSpec((N, K), lambda i: (0, 0)),
        ],
        out_specs=pl.BlockSpec(
            (1, 1), lambda i: (0, 0), memory_space=pltpu.SMEM),
        out_shape=jax.ShapeDtypeStruct((1, 1), jnp.float32),
        scratch_shapes=[pltpu.VMEM((KA, N), jnp.float32)],
        compiler_params=pltpu.CompilerParams(
            dimension_semantics=("arbitrary",)),
    )(outputs, targets)
    return res[0, 0]
